# baseline (speedup 1.0000x reference)
import functools

import jax
import jax.numpy as jnp
from jax import lax
from jax.experimental import pallas as pl
from jax.experimental.pallas import tpu as pltpu
from jax.experimental.pallas import tpu_sc as plsc

GROUPS = [
    [0, 1], [1, 2, 3, 4, 5], [2, 3, 6], [3, 6, 7], [6, 7], [2, 4, 8],
    [4, 8, 9], [8, 9], [10, 11, 12], [11, 12, 13], [12, 13], [10, 14, 15],
    [14, 15, 16], [15, 16], [5, 10, 11, 14], [2, 5, 10], [0, 1, 2],
]

_B, _J, _C, _D = 16384, 17, 3, 32
_JC = _J * _C
_ROW = _JC * _D
_O = 3 * len(GROUPS)

_MEMBERS = []
_m = 0
for _i, _g in enumerate(GROUPS):
    for _k, _j in enumerate(_g):
        for _c in range(_C):
            _MEMBERS.append((_m, 3 * _j + _c, 3 * _i + _c))
            _m += 1
_NW = _m

_NWORK = 32
_RPW = _B // _NWORK
_CH = 16
_NCH = _RPW // _CH
_NBUF = 3


def _pack_tables(weights, biases):
    w_flat = jnp.concatenate([w.reshape(-1) for w in weights])
    bias_sum = jnp.concatenate([jnp.sum(b, axis=0) for b in biases])
    tab = jnp.concatenate([w_flat, bias_sum])
    return jnp.repeat(tab[:, None], _CH, axis=1)


def _make_sc_kernel():
    mesh = plsc.VectorSubcoreMesh(core_axis_name="c", subcore_axis_name="s")
    scratch = (
        [pltpu.VMEM((_CH * _ROW,), jnp.float32) for _ in range(_NBUF)]
        + [pltpu.VMEM((_CH * _O,), jnp.float32) for _ in range(_NBUF)]
        + [pltpu.VMEM((_NW + _O, _CH), jnp.float32)]
        + [pltpu.VMEM((_JC * _CH,), jnp.float32)]
        + [pltpu.SemaphoreType.DMA for _ in range(2 * _NBUF)]
    )

    @functools.partial(
        pl.kernel,
        mesh=mesh,
        out_type=jax.ShapeDtypeStruct((_B * _O,), jnp.float32),
        scratch_types=scratch,
        compiler_params=pltpu.CompilerParams(needs_layout_passes=False),
    )
    def k(x_hbm, tab_hbm, out_hbm, *refs):
        ibufs = refs[0:_NBUF]
        obufs = refs[_NBUF:2 * _NBUF]
        tab_v = refs[2 * _NBUF]
        xi = refs[2 * _NBUF + 1]
        isems = refs[2 * _NBUF + 2:2 * _NBUF + 2 + _NBUF]
        osems = refs[2 * _NBUF + 2 + _NBUF:]

        wid = lax.axis_index("s") * 2 + lax.axis_index("c")
        base_row = wid * _RPW

        pltpu.sync_copy(tab_hbm, tab_v)

        iota = lax.iota(jnp.int32, _CH)
        rowv = iota * _ROW
        outv = iota * _O

        def in_slice(g):
            start = (base_row + g * _CH) * _ROW
            return x_hbm.at[pl.ds(start, _CH * _ROW)]

        def out_slice(g):
            start = (base_row + g * _CH) * _O
            return out_hbm.at[pl.ds(start, _CH * _O)]

        for b in range(_NBUF):
            pltpu.async_copy(in_slice(b), ibufs[b], isems[b])

        def maybe(pred, fn):
            if isinstance(pred, bool):
                if pred:
                    fn()
            else:
                pl.when(pred)(fn)

        def chunk_step(g, b, out_wait_pred, refill_pred):
            fbuf, obuf = ibufs[b], obufs[b]
            isem, osem = isems[b], osems[b]
            pltpu.make_async_copy(in_slice(g), fbuf, isem).wait()

            def jc_step(t, carry):
                for u in range(3):
                    jc = t * 3 + u
                    basev = rowv + jc * _D
                    acc = plsc.load_gather(fbuf, [basev])
                    for d in range(1, _D):
                        acc = acc + plsc.load_gather(fbuf, [basev + d])
                    plsc.store_scatter(xi, [iota + jc * _CH], acc)
                return carry

            lax.fori_loop(0, _JC // 3, jc_step, 0)

            maybe(out_wait_pred,
                  lambda: pltpu.make_async_copy(obuf, out_slice(g), osem).wait())

            accs = {}
            for (m, jc, o) in _MEMBERS:
                w = tab_v[m, :]
                v = xi[pl.ds(jc * _CH, _CH)]
                if o in accs:
                    accs[o] = accs[o] + w * v
                else:
                    accs[o] = tab_v[_NW + o, :] + w * v
            for o in range(_O):
                plsc.store_scatter(obuf, [outv + o], accs[o])

            pltpu.async_copy(obuf, out_slice(g), osem)

            def _refill():
                pltpu.async_copy(in_slice(g + _NBUF), fbuf, isem)

            maybe(refill_pred, _refill)

        n_rounds = _NCH // _NBUF
        rem = _NCH - n_rounds * _NBUF

        def ring_step(c, carry):
            for b in range(_NBUF):
                g = c * _NBUF + b
                chunk_step(g, b, c > 0, g + _NBUF < _NCH)
            return carry

        lax.fori_loop(0, n_rounds, ring_step, 0)

        for r in range(rem):
            g = n_rounds * _NBUF + r
            chunk_step(g, g % _NBUF, True, False)

        for g in range(_NCH - _NBUF, _NCH):
            b = g % _NBUF
            pltpu.make_async_copy(obufs[b], out_slice(g), osems[b]).wait()

    return k


_SC_KERNEL = _make_sc_kernel()


@jax.jit
def _run_sc(x_flat, tab):
    out_flat = _SC_KERNEL(x_flat, tab)
    return out_flat.reshape(_B, _O)


def kernel(input, weights, biases):
    tab = _pack_tables(weights, biases)
    x_flat = input.reshape(-1)
    return _run_sc(x_flat, tab)


# speedup vs baseline: 4.1294x; 4.1294x over previous
import numpy as np

import jax
import jax.numpy as jnp
from jax.experimental import pallas as pl

GROUPS = [
    [0, 1], [1, 2, 3, 4, 5], [2, 3, 6], [3, 6, 7], [6, 7], [2, 4, 8],
    [4, 8, 9], [8, 9], [10, 11, 12], [11, 12, 13], [12, 13], [10, 14, 15],
    [14, 15, 16], [15, 16], [5, 10, 11, 14], [2, 5, 10], [0, 1, 2],
]

_B, _J, _C, _D = 16384, 17, 3, 32
_K = _J * _C * _D
_O = 3 * len(GROUPS)

_ROWS, _COLS = [], []
for _i, _g in enumerate(GROUPS):
    for _k, _j in enumerate(_g):
        for _c in range(_C):
            for _d in range(_D):
                _ROWS.append((3 * _j + _c) * _D + _d)
                _COLS.append(3 * _i + _c)
_ROWS = np.asarray(_ROWS, dtype=np.int32)
_COLS = np.asarray(_COLS, dtype=np.int32)

_BT = 1024


def _pack_m(weights, biases):
    w_flat = jnp.concatenate([w.reshape(-1) for w in weights])
    m = jnp.zeros((_K, _O), jnp.float32).at[_ROWS, _COLS].add(
        jnp.repeat(w_flat, _D))
    bias_row = jnp.concatenate([jnp.sum(b, axis=0) for b in biases])
    return m, bias_row.reshape(1, _O)


def _tc_body(x_ref, m_ref, b_ref, o_ref):
    o_ref[...] = (
        jnp.dot(x_ref[...], m_ref[...], preferred_element_type=jnp.float32)
        + b_ref[...]
    )


@jax.jit
def _run_tc(x_flat, m, bias_row):
    return pl.pallas_call(
        _tc_body,
        grid=(_B // _BT,),
        in_specs=[
            pl.BlockSpec((_BT, _K), lambda i: (i, 0)),
            pl.BlockSpec((_K, _O), lambda i: (0, 0)),
            pl.BlockSpec((1, _O), lambda i: (0, 0)),
        ],
        out_specs=pl.BlockSpec((_BT, _O), lambda i: (i, 0)),
        out_shape=jax.ShapeDtypeStruct((_B, _O), jnp.float32),
    )(x_flat, m, bias_row)


def kernel(input, weights, biases):
    m, bias_row = _pack_m(weights, biases)
    x_flat = input.reshape(_B, _K)
    return _run_tc(x_flat, m, bias_row)
